# baseline (speedup 1.0000x reference)
import functools

import jax
import jax.numpy as jnp
from jax import lax
from jax.experimental import pallas as pl
from jax.experimental.pallas import tpu as pltpu
from jax.experimental.pallas import tpu_sc as plsc

_FEAT_STRIDE = 16.0

_B = 16
_A = 9
_K = 4096
_ELEMS = _K * _A * 4
_NTILES = 24
_CHUNK = _ELEMS // _NTILES
_NVEC = _CHUNK // 16


def _anchor_vals(flat):
    g = lax.shift_right_logical(flat, 9)
    c = lax.bitwise_and(lax.shift_right_logical(flat, 7), 3)
    n = lax.bitwise_or(lax.shift_left(g, 7), lax.bitwise_and(flat, 127))
    k = lax.div(n, 9)
    a = lax.rem(n, 9)
    ri = lax.div(a, 3)
    si = lax.rem(a, 3)
    ws = jnp.where(ri == 0, 23.0, jnp.where(ri == 1, 16.0, 11.0))
    hs = jnp.where(ri == 0, 12.0, jnp.where(ri == 1, 16.0, 22.0))
    sc = jnp.where(si == 0, 8.0, jnp.where(si == 1, 16.0, 32.0))
    hw = 0.5 * (ws * sc - 1.0)
    hh = 0.5 * (hs * sc - 1.0)
    base = jnp.where(c == 0, 7.5 - hw,
                     jnp.where(c == 1, 7.5 - hh,
                               jnp.where(c == 2, 7.5 + hw, 7.5 + hh)))
    x = lax.bitwise_and(k, 63).astype(jnp.float32)
    y = lax.shift_right_logical(k, 6).astype(jnp.float32)
    c_even = lax.bitwise_and(c, 1) == 0
    return base + _FEAT_STRIDE * jnp.where(c_even, x, y)


_ROWS = _CHUNK // 128


@functools.partial(
    pl.kernel,
    mesh=plsc.VectorSubcoreMesh(core_axis_name="c", subcore_axis_name="s"),
    out_type=jax.ShapeDtypeStruct((_B, _ELEMS // 128, 128), jnp.float32),
    scratch_types=[
        pltpu.VMEM((_ROWS, 128), jnp.float32),
        pltpu.SemaphoreType.DMA,
    ],
)
def _anchors_sc(out_hbm, chunk_v, sem):
    tile = lax.axis_index("s") * 2 + lax.axis_index("c")

    @pl.when(tile < _NTILES)
    def _():
        base = tile * _CHUNK
        lane = lax.iota(jnp.int32, 16)

        def fill(q, _):
            r = lax.shift_right_logical(q, 3)
            col = lax.bitwise_and(q, 7) * 16
            chunk_v[r, pl.ds(col, 16)] = _anchor_vals(base + q * 16 + lane)
            return _

        lax.fori_loop(0, _NVEC, fill, None)
        row0 = tile * _ROWS
        for grp in range(0, _B, 4):
            copies = [
                pltpu.async_copy(
                    chunk_v, out_hbm.at[b, pl.ds(row0, _ROWS), :], sem)
                for b in range(grp, grp + 4)
            ]
            for cp in copies:
                cp.wait()


def _fg_body(scores_ref, bbox_ref, fg_ref, bbox_out_ref, sem):
    @pl.when(pl.program_id(0) == 0)
    def _():
        pltpu.make_async_copy(bbox_ref, bbox_out_ref, sem).start()

    fg_ref[...] = scores_ref[...]

    @pl.when(pl.program_id(0) == pl.num_programs(0) - 1)
    def _():
        pltpu.make_async_copy(bbox_ref, bbox_out_ref, sem).wait()


def kernel(scores, bbox_deltas, im_info, cfg_key):
    B = scores.shape[0]
    A = 9
    H, W = scores.shape[2], scores.shape[3]
    K = H * W

    anc = _anchors_sc()

    fg, bbox_out = pl.pallas_call(
        _fg_body,
        grid=(B,),
        in_specs=[
            pl.BlockSpec((1, A, H, W), lambda b: (b, 1, 0, 0)),
            pl.BlockSpec(memory_space=pl.ANY),
        ],
        out_specs=[
            pl.BlockSpec((1, A, H, W), lambda b: (b, 0, 0, 0)),
            pl.BlockSpec(memory_space=pl.ANY),
        ],
        out_shape=[
            jax.ShapeDtypeStruct((B, A, H, W), jnp.float32),
            jax.ShapeDtypeStruct(bbox_deltas.shape, bbox_deltas.dtype),
        ],
        scratch_shapes=[pltpu.SemaphoreType.DMA],
        compiler_params=pltpu.CompilerParams(
            dimension_semantics=("arbitrary",),
        ),
    )(scores, bbox_deltas)

    anchors = (anc.reshape(B, (K * A) // 128, 4, 128)
               .transpose(0, 1, 3, 2)
               .reshape(B, K * A, 4))
    return (fg, bbox_out, im_info, anchors)


# speedup vs baseline: 24.4536x; 24.4536x over previous
import jax
import jax.numpy as jnp
from jax.experimental import pallas as pl
from jax.experimental.pallas import tpu as pltpu

_FEAT_STRIDE = 16.0


def _anchor_pattern():
    r = jax.lax.broadcasted_iota(jnp.int32, (1152, 128), 0)
    l = jax.lax.broadcasted_iota(jnp.int32, (1152, 128), 1)
    rf = r.astype(jnp.float32)
    g = jnp.floor(rf * 0.25)
    c = rf - 4.0 * g
    n = g * 128.0 + l.astype(jnp.float32)
    k = jnp.floor((n + 0.5) * (1.0 / 9.0))
    a = n - 9.0 * k
    ri = jnp.floor((a + 0.5) * (1.0 / 3.0))
    si = a - 3.0 * ri
    ws = jnp.where(ri < 0.5, 23.0, jnp.where(ri < 1.5, 16.0, 11.0))
    hs = jnp.where(ri < 0.5, 12.0, jnp.where(ri < 1.5, 16.0, 22.0))
    sc = jnp.where(si < 0.5, 8.0, jnp.where(si < 1.5, 16.0, 32.0))
    hw = 0.5 * (ws * sc - 1.0)
    hh = 0.5 * (hs * sc - 1.0)
    base = jnp.where(c < 0.5, 7.5 - hw,
                     jnp.where(c < 1.5, 7.5 - hh,
                               jnp.where(c < 2.5, 7.5 + hw, 7.5 + hh)))
    y = jnp.floor(k * (1.0 / 64.0))
    x = k - 64.0 * y
    c_even = jnp.logical_or(c < 0.5, jnp.abs(c - 2.0) < 0.5)
    return base + _FEAT_STRIDE * jnp.where(c_even, x, y)


def _body(scores_ref, bbox_ref, fg_ref, bbox_out_ref, anc_ref, pat_ref):
    @pl.when(pl.program_id(0) == 0)
    def _():
        pat_ref[...] = _anchor_pattern()

    fg_ref[...] = scores_ref[...]
    bbox_out_ref[...] = bbox_ref[...]
    anc_ref[0] = pat_ref[...]


def kernel(scores, bbox_deltas, im_info, cfg_key):
    B = scores.shape[0]
    A = 9
    H, W = scores.shape[2], scores.shape[3]
    K = H * W

    C = bbox_deltas.shape[1]
    fg, bbox_out, anc = pl.pallas_call(
        _body,
        grid=(B,),
        in_specs=[
            pl.BlockSpec((1, A, H, W), lambda b: (b, 1, 0, 0)),
            pl.BlockSpec((1, C, H, W), lambda b: (b, 0, 0, 0)),
        ],
        out_specs=[
            pl.BlockSpec((1, A, H, W), lambda b: (b, 0, 0, 0)),
            pl.BlockSpec((1, C, H, W), lambda b: (b, 0, 0, 0)),
            pl.BlockSpec((1, (K * A * 4) // 128, 128), lambda b: (b, 0, 0)),
        ],
        out_shape=[
            jax.ShapeDtypeStruct((B, A, H, W), jnp.float32),
            jax.ShapeDtypeStruct((B, C, H, W), jnp.float32),
            jax.ShapeDtypeStruct((B, (K * A * 4) // 128, 128), jnp.float32),
        ],
        scratch_shapes=[pltpu.VMEM(((K * A * 4) // 128, 128), jnp.float32)],
        compiler_params=pltpu.CompilerParams(
            dimension_semantics=("arbitrary",),
        ),
    )(scores, bbox_deltas)

    anchors = (anc.reshape(B, (K * A) // 128, 4, 128)
               .transpose(0, 1, 3, 2)
               .reshape(B, K * A, 4))
    return (fg, bbox_out, im_info, anchors)
